# baseline (speedup 1.0000x reference)
import jax
import jax.numpy as jnp
from jax import lax
from jax.experimental import pallas as pl
from jax.experimental.pallas import tpu as pltpu
from jax.experimental.pallas import tpu_sc as plsc

D = 64
B, L = 64, 2048
N = B * L
N_T, N_B, N_P, N_K, N_D = 64, 128, 128, 256, 128

NC, NS = 2, 16
NW = NC * NS
TPW = N // NW
IG = 128
CHUNK = 128
NCHUNK = TPW // CHUNK
CPB = L // CHUNK


def _project_body(tt, bt, pt, kt, dt, w, b, otb, opd, otok):
    bias = b[0, :]
    p_t = jnp.dot(tt[...], w[0:64, :], preferred_element_type=jnp.float32) + bias
    p_b = jnp.dot(bt[...], w[64:128, :], preferred_element_type=jnp.float32)
    p_p = jnp.dot(pt[...], w[128:192, :], preferred_element_type=jnp.float32)
    p_k = jnp.dot(kt[...], w[192:256, :], preferred_element_type=jnp.float32)
    p_d = jnp.dot(dt[...], w[256:320, :], preferred_element_type=jnp.float32)
    for i in range(N_T):
        otb[pl.ds(i * N_B, N_B), :] = p_b + p_t[i:i + 1, :]
    for i in range(N_P):
        opd[pl.ds(i * N_D, N_D), :] = p_d + p_p[i:i + 1, :]
    otok[...] = p_k


def _project(tt, bt, pt, kt, dt, w, b):
    return pl.pallas_call(
        _project_body,
        out_shape=[
            jax.ShapeDtypeStruct((N_T * N_B, D), jnp.float32),
            jax.ShapeDtypeStruct((N_P * N_D, D), jnp.float32),
            jax.ShapeDtypeStruct((N_K, D), jnp.float32),
        ],
    )(tt, bt, pt, kt, dt, w, b.reshape(1, D))


def _lookup_body(ttb, tpd, ttok, iall, out,
                 vi0, vi1, ci0, ci1, rg0, rg1, rb0, rb1, rc0, rc1,
                 ac0, ac1, sidx, sg, so):
    cid = lax.axis_index("c")
    sid = lax.axis_index("s")
    wid = sid * NC + cid
    row0 = wid * NCHUNK
    vi = (vi0, vi1)
    ci = (ci0, ci1)
    rg = (rg0, rg1)
    rb = (rb0, rb1)
    rc = (rc0, rc1)
    ac = (ac0, ac1)

    def idx_cp(k):
        return pltpu.make_async_copy(
            iall.at[pl.ds(row0 + k, 1)], vi[k & 1], sidx)

    def gather_cps(k):
        p = k & 1
        return [
            pltpu.make_async_copy(ttb.at[ci[p].at[0, 0]], rg[p], sg),
            pltpu.make_async_copy(tpd.at[ci[p].at[0, 1]], rb[p], sg),
            pltpu.make_async_copy(ttok.at[vi[p].at[0, 3]], rc[p], sg),
        ]

    def out_cp(k):
        bi = 2 * wid + (k // CPB)
        l0 = (k % CPB) * CHUNK
        return pltpu.make_async_copy(
            ac[k & 1], out.at[bi, :, pl.ds(l0, CHUNK)], so)

    def combine(k):
        p = k & 1
        vip, cip = vi[p], ci[p]

        def cb(j, c):
            sl = pl.ds(j * 16, 16)
            cip[0, 0, sl] = vip[0, 0, sl] * N_B + vip[0, 1, sl]
            cip[0, 1, sl] = vip[0, 2, sl] * N_D + vip[0, 4, sl]
            return c
        lax.fori_loop(0, IG // 16, cb, 0)

    def accum(k):
        p = k & 1
        rgp, rbp, rcp, acp = rg[p], rb[p], rc[p], ac[p]
        rows = [lax.broadcasted_iota(jnp.int32, (16,), 0) + lb * 16
                for lb in range(CHUNK // 16)]

        def ab(d, c):
            cols = jnp.zeros((16,), jnp.int32) + d
            for lb in range(CHUNK // 16):
                v = (plsc.load_gather(rgp, [rows[lb], cols])
                     + plsc.load_gather(rbp, [rows[lb], cols])
                     + plsc.load_gather(rcp, [rows[lb], cols]))
                acp[d, pl.ds(lb * 16, 16)] = v
            return c
        lax.fori_loop(0, D, ab, 0)

    idx_cp(0).start()
    idx_cp(0).wait()
    combine(0)
    for cp in gather_cps(0):
        cp.start()
    if NCHUNK > 1:
        idx_cp(1).start()

    for k in range(NCHUNK):
        if k + 1 < NCHUNK:
            idx_cp(k + 1).wait()
            combine(k + 1)
        for cp in gather_cps(k):
            cp.wait()
        if k >= 1:
            out_cp(k - 1).wait()
        if k + 1 < NCHUNK:
            for cp in gather_cps(k + 1):
                cp.start()
            if k + 2 < NCHUNK:
                idx_cp(k + 2).start()
        accum(k)
        out_cp(k).start()
    out_cp(NCHUNK - 1).wait()


def _lookup(ttb, tpd, ttok, iall):
    mesh = plsc.VectorSubcoreMesh(core_axis_name="c", subcore_axis_name="s")
    f = pl.kernel(
        _lookup_body,
        out_type=jax.ShapeDtypeStruct((B, D, L), jnp.float32),
        mesh=mesh,
        scratch_types=[
            pltpu.VMEM((1, 5, IG), jnp.int32),
            pltpu.VMEM((1, 5, IG), jnp.int32),
            pltpu.VMEM((1, 2, IG), jnp.int32),
            pltpu.VMEM((1, 2, IG), jnp.int32),
            pltpu.VMEM((CHUNK, D), jnp.float32),
            pltpu.VMEM((CHUNK, D), jnp.float32),
            pltpu.VMEM((CHUNK, D), jnp.float32),
            pltpu.VMEM((CHUNK, D), jnp.float32),
            pltpu.VMEM((CHUNK, D), jnp.float32),
            pltpu.VMEM((CHUNK, D), jnp.float32),
            pltpu.VMEM((D, CHUNK), jnp.float32),
            pltpu.VMEM((D, CHUNK), jnp.float32),
            pltpu.SemaphoreType.DMA,
            pltpu.SemaphoreType.DMA,
            pltpu.SemaphoreType.DMA,
        ],
        compiler_params=pltpu.CompilerParams(use_tc_tiling_on_sc=False,
                                             needs_layout_passes=False),
    )
    return f(ttb, tpd, ttok, iall)


def kernel(Tempo, Bar, Position, Token, Duration, tempo_table, bar_table,
           pos_table, token_table, dur_table, W_dec, b_dec):
    ttb, tpd, ttok = _project(tempo_table, bar_table, pos_table,
                              token_table, dur_table, W_dec, b_dec)
    iall = (jnp.stack([Tempo.reshape(N), Bar.reshape(N), Position.reshape(N),
                       Token.reshape(N), Duration.reshape(N)])
            .reshape(5, N // IG, IG).transpose(1, 0, 2))
    out = _lookup(ttb, tpd, ttok, iall)
    return out.transpose(0, 2, 1)


# speedup vs baseline: 2.8811x; 2.8811x over previous
import jax
import jax.numpy as jnp
from jax import lax
from jax.experimental import pallas as pl
from jax.experimental.pallas import tpu as pltpu
from jax.experimental.pallas import tpu_sc as plsc

D = 64
B, L = 64, 2048
N = B * L
N_T, N_B, N_P, N_K, N_D = 64, 128, 128, 256, 128

NC, NS = 2, 16
NW = NC * NS
TPW = N // NW
IG = 128
CHUNK = 128
NCHUNK = TPW // CHUNK
CPB = L // CHUNK


def _project_body(tt, bt, pt, kt, dt, w, b, otb, opd, otok):
    bias = b[0, :]
    p_t = jnp.dot(tt[...], w[0:64, :], preferred_element_type=jnp.float32) + bias
    p_b = jnp.dot(bt[...], w[64:128, :], preferred_element_type=jnp.float32)
    p_p = jnp.dot(pt[...], w[128:192, :], preferred_element_type=jnp.float32)
    p_k = jnp.dot(kt[...], w[192:256, :], preferred_element_type=jnp.float32)
    p_d = jnp.dot(dt[...], w[256:320, :], preferred_element_type=jnp.float32)
    for i in range(N_T):
        otb[pl.ds(i * N_B, N_B), :] = p_b + p_t[i:i + 1, :]
    for i in range(N_P):
        opd[pl.ds(i * N_D, N_D), :] = p_d + p_p[i:i + 1, :]
    otok[...] = p_k


def _project(tt, bt, pt, kt, dt, w, b):
    return pl.pallas_call(
        _project_body,
        out_shape=[
            jax.ShapeDtypeStruct((N_T * N_B, D), jnp.float32),
            jax.ShapeDtypeStruct((N_P * N_D, D), jnp.float32),
            jax.ShapeDtypeStruct((N_K, D), jnp.float32),
        ],
    )(tt, bt, pt, kt, dt, w, b.reshape(1, D))


def _lookup_body(ttb, tpd, ttok, iall, out,
                 vi0, vi1, ci0, ci1, rg0, rg1, rb0, rb1, rc0, rc1,
                 ac0, ac1, sidx, sg, so):
    cid = lax.axis_index("c")
    sid = lax.axis_index("s")
    wid = sid * NC + cid
    row0 = wid * NCHUNK
    vi = (vi0, vi1)
    ci = (ci0, ci1)
    rg = (rg0, rg1)
    rb = (rb0, rb1)
    rc = (rc0, rc1)
    ac = (ac0, ac1)

    def idx_cp(k, p):
        return pltpu.make_async_copy(
            iall.at[pl.ds(row0 + k, 1)], vi[p], sidx)

    def gather_cps(k, p):
        return [
            pltpu.make_async_copy(ttb.at[ci[p].at[0, 0]], rg[p], sg),
            pltpu.make_async_copy(tpd.at[ci[p].at[0, 1]], rb[p], sg),
            pltpu.make_async_copy(ttok.at[vi[p].at[0, 3]], rc[p], sg),
        ]

    def out_cp(k, p):
        bi = 2 * wid + (k // CPB)
        l0 = (k % CPB) * CHUNK
        return pltpu.make_async_copy(
            ac[p], out.at[bi, :, pl.ds(l0, CHUNK)], so)

    def combine(k, p):
        vip, cip = vi[p], ci[p]

        def cb(j, c):
            sl = pl.ds(j * 16, 16)
            cip[0, 0, sl] = vip[0, 0, sl] * N_B + vip[0, 1, sl]
            cip[0, 1, sl] = vip[0, 2, sl] * N_D + vip[0, 4, sl]
            return c
        lax.fori_loop(0, IG // 16, cb, 0)

    jbase = lax.broadcasted_iota(jnp.int32, (16,), 0)
    rots = [lax.rem(jbase + i, 16) for i in range(16)]
    NLT = CHUNK // 16

    def accum(p):
        rgp, rbp, rcp, acp = rg[p], rb[p], rc[p], ac[p]

        def ab(q, c):
            l0 = lax.rem(q, NLT) * 16
            d0 = lax.div(q, NLT) * 16
            didx = jbase + d0
            for i in range(16):
                ridx = rots[i] + l0
                v = (plsc.load_gather(rgp, [ridx, didx])
                     + plsc.load_gather(rbp, [ridx, didx])
                     + plsc.load_gather(rcp, [ridx, didx]))
                plsc.store_scatter(acp, [didx, ridx], v)
            return c
        lax.fori_loop(0, NLT * (D // 16), ab, 0)

    def chunk_step(k, p, first, has_next, has_next2):
        if has_next:
            idx_cp(k + 1, p ^ 1).wait()
            combine(k + 1, p ^ 1)
        for cp in gather_cps(k, p):
            cp.wait()
        if not first:
            out_cp(k - 1, p ^ 1).wait()
        if has_next:
            for cp in gather_cps(k + 1, p ^ 1):
                cp.start()
            if has_next2:
                idx_cp(k + 2, p).start()
        accum(p)
        out_cp(k, p).start()

    idx_cp(0, 0).start()
    idx_cp(0, 0).wait()
    combine(0, 0)
    for cp in gather_cps(0, 0):
        cp.start()
    idx_cp(1, 1).start()

    chunk_step(0, 0, True, True, True)
    chunk_step(1, 1, False, True, True)

    def steady(j, c):
        k0 = 2 + 2 * j
        chunk_step(k0, 0, False, True, True)
        chunk_step(k0 + 1, 1, False, True, True)
        return c
    lax.fori_loop(0, (NCHUNK - 4) // 2, steady, 0)

    chunk_step(NCHUNK - 2, 0, False, True, False)
    chunk_step(NCHUNK - 1, 1, False, False, False)
    out_cp(NCHUNK - 1, 1).wait()


def _lookup(ttb, tpd, ttok, iall):
    mesh = plsc.VectorSubcoreMesh(core_axis_name="c", subcore_axis_name="s")
    f = pl.kernel(
        _lookup_body,
        out_type=jax.ShapeDtypeStruct((B, D, L), jnp.float32),
        mesh=mesh,
        scratch_types=[
            pltpu.VMEM((1, 5, IG), jnp.int32),
            pltpu.VMEM((1, 5, IG), jnp.int32),
            pltpu.VMEM((1, 2, IG), jnp.int32),
            pltpu.VMEM((1, 2, IG), jnp.int32),
            pltpu.VMEM((CHUNK, D), jnp.float32),
            pltpu.VMEM((CHUNK, D), jnp.float32),
            pltpu.VMEM((CHUNK, D), jnp.float32),
            pltpu.VMEM((CHUNK, D), jnp.float32),
            pltpu.VMEM((CHUNK, D), jnp.float32),
            pltpu.VMEM((CHUNK, D), jnp.float32),
            pltpu.VMEM((D, CHUNK), jnp.float32),
            pltpu.VMEM((D, CHUNK), jnp.float32),
            pltpu.SemaphoreType.DMA,
            pltpu.SemaphoreType.DMA,
            pltpu.SemaphoreType.DMA,
        ],
        compiler_params=pltpu.CompilerParams(use_tc_tiling_on_sc=False,
                                             needs_layout_passes=False),
    )
    return f(ttb, tpd, ttok, iall)


def kernel(Tempo, Bar, Position, Token, Duration, tempo_table, bar_table,
           pos_table, token_table, dur_table, W_dec, b_dec):
    ttb, tpd, ttok = _project(tempo_table, bar_table, pos_table,
                              token_table, dur_table, W_dec, b_dec)
    iall = (jnp.stack([Tempo.reshape(N), Bar.reshape(N), Position.reshape(N),
                       Token.reshape(N), Duration.reshape(N)])
            .reshape(5, N // IG, IG).transpose(1, 0, 2))
    out = _lookup(ttb, tpd, ttok, iall)
    return out.transpose(0, 2, 1)


# speedup vs baseline: 3.0001x; 1.0413x over previous
import jax
import jax.numpy as jnp
from jax import lax
from jax.experimental import pallas as pl
from jax.experimental.pallas import tpu as pltpu
from jax.experimental.pallas import tpu_sc as plsc

D = 64
B, L = 64, 2048
N = B * L
N_T, N_B, N_P, N_K, N_D = 64, 128, 128, 256, 128

NC, NS = 2, 16
NW = NC * NS
TPW = N // NW
IG = 128
CHUNK = 256
NG = CHUNK // IG
NCHUNK = TPW // CHUNK
CPB = L // CHUNK


def _project_body(tt, bt, pt, kt, dt, w, b, otb, opd, otok):
    bias = b[0, :]
    p_t = jnp.dot(tt[...], w[0:64, :], preferred_element_type=jnp.float32) + bias
    p_b = jnp.dot(bt[...], w[64:128, :], preferred_element_type=jnp.float32)
    p_p = jnp.dot(pt[...], w[128:192, :], preferred_element_type=jnp.float32)
    p_k = jnp.dot(kt[...], w[192:256, :], preferred_element_type=jnp.float32)
    p_d = jnp.dot(dt[...], w[256:320, :], preferred_element_type=jnp.float32)
    for i in range(N_T):
        otb[pl.ds(i * N_B, N_B), :] = p_b + p_t[i:i + 1, :]
    for i in range(N_P):
        opd[pl.ds(i * N_D, N_D), :] = p_d + p_p[i:i + 1, :]
    otok[...] = p_k


def _project(tt, bt, pt, kt, dt, w, b):
    return pl.pallas_call(
        _project_body,
        out_shape=[
            jax.ShapeDtypeStruct((N_T * N_B, D), jnp.float32),
            jax.ShapeDtypeStruct((N_P * N_D, D), jnp.float32),
            jax.ShapeDtypeStruct((N_K, D), jnp.float32),
        ],
    )(tt, bt, pt, kt, dt, w, b.reshape(1, D))


def _lookup_body(ttb, tpd, ttok, i0, i1, i2, i3, i4, out,
                 vi0, vi1, ci0, ci1, rg0, rg1, rb0, rb1, rc0, rc1,
                 ac0, sidx, sg, so):
    cid = lax.axis_index("c")
    sid = lax.axis_index("s")
    wid = sid * NC + cid
    row0 = wid * (TPW // IG)
    vi = (vi0, vi1)
    ci = (ci0, ci1)
    rg = (rg0, rg1)
    rb = (rb0, rb1)
    rc = (rc0, rc1)
    ac = (ac0, ac0)
    idx_arrs = (i0, i1, i2, i3, i4)

    def idx_cps(k, p):
        r = pl.ds(row0 + k * NG, NG)
        return [pltpu.make_async_copy(idx_arrs[f].at[r], vi[p].at[f], sidx)
                for f in range(5)]

    def gather_cps(k, p):
        cps = []
        for g in range(NG):
            dst = pl.ds(g * IG, IG)
            cps.append(pltpu.make_async_copy(
                ttb.at[ci[p].at[g, 0]], rg[p].at[dst], sg))
            cps.append(pltpu.make_async_copy(
                tpd.at[ci[p].at[g, 1]], rb[p].at[dst], sg))
            cps.append(pltpu.make_async_copy(
                ttok.at[vi[p].at[3, g]], rc[p].at[dst], sg))
        return cps

    def out_cp(k, p):
        bi = 2 * wid + (k // CPB)
        l0 = (k % CPB) * CHUNK
        return pltpu.make_async_copy(
            ac[p], out.at[bi, :, pl.ds(l0, CHUNK)], so)

    def combine(k, p):
        vip, cip = vi[p], ci[p]

        def cb(j, c):
            g = j // (IG // 16)
            col = (j % (IG // 16)) * 16
            sl = pl.ds(col, 16)
            cip[g, 0, sl] = vip[0, g, sl] * N_B + vip[1, g, sl]
            cip[g, 1, sl] = vip[2, g, sl] * N_D + vip[4, g, sl]
            return c
        lax.fori_loop(0, NG * (IG // 16), cb, 0)

    jbase = lax.broadcasted_iota(jnp.int32, (16,), 0)
    rots = [lax.rem(jbase + i, 16) for i in range(16)]
    NLT = CHUNK // 16

    def accum(p):
        rgp, rbp, rcp, acp = rg[p], rb[p], rc[p], ac[p]

        def ab(q, c):
            l0 = lax.rem(q, NLT) * 16
            d0 = lax.div(q, NLT) * 16
            didx = jbase + d0
            for i in range(16):
                ridx = rots[i] + l0
                v = (plsc.load_gather(rgp, [ridx, didx])
                     + plsc.load_gather(rbp, [ridx, didx])
                     + plsc.load_gather(rcp, [ridx, didx]))
                plsc.store_scatter(acp, [didx, ridx], v)
            return c
        lax.fori_loop(0, NLT * (D // 16), ab, 0)

    def chunk_step(k, p, first, has_next, has_next2):
        if has_next:
            for cp in idx_cps(k + 1, p ^ 1):
                cp.wait()
            combine(k + 1, p ^ 1)
        for cp in gather_cps(k, p):
            cp.wait()
        if not first:
            out_cp(k - 1, p ^ 1).wait()
        if has_next:
            for cp in gather_cps(k + 1, p ^ 1):
                cp.start()
            if has_next2:
                for cp in idx_cps(k + 2, p):
                    cp.start()
        accum(p)
        out_cp(k, p).start()

    for cp in idx_cps(0, 0):
        cp.start()
    for cp in idx_cps(0, 0):
        cp.wait()
    combine(0, 0)
    for cp in gather_cps(0, 0):
        cp.start()
    for cp in idx_cps(1, 1):
        cp.start()

    chunk_step(0, 0, True, True, True)
    chunk_step(1, 1, False, True, True)

    def steady(j, c):
        k0 = 2 + 2 * j
        chunk_step(k0, 0, False, True, True)
        chunk_step(k0 + 1, 1, False, True, True)
        return c
    lax.fori_loop(0, (NCHUNK - 4) // 2, steady, 0)

    chunk_step(NCHUNK - 2, 0, False, True, False)
    chunk_step(NCHUNK - 1, 1, False, False, False)
    out_cp(NCHUNK - 1, 1).wait()


def _lookup(ttb, tpd, ttok, i0, i1, i2, i3, i4):
    mesh = plsc.VectorSubcoreMesh(core_axis_name="c", subcore_axis_name="s")
    f = pl.kernel(
        _lookup_body,
        out_type=jax.ShapeDtypeStruct((B, D, L), jnp.float32),
        mesh=mesh,
        scratch_types=[
            pltpu.VMEM((5, NG, IG), jnp.int32),
            pltpu.VMEM((5, NG, IG), jnp.int32),
            pltpu.VMEM((NG, 2, IG), jnp.int32),
            pltpu.VMEM((NG, 2, IG), jnp.int32),
            pltpu.VMEM((CHUNK, D), jnp.float32),
            pltpu.VMEM((CHUNK, D), jnp.float32),
            pltpu.VMEM((CHUNK, D), jnp.float32),
            pltpu.VMEM((CHUNK, D), jnp.float32),
            pltpu.VMEM((CHUNK, D), jnp.float32),
            pltpu.VMEM((CHUNK, D), jnp.float32),
            pltpu.VMEM((D, CHUNK), jnp.float32),
            pltpu.SemaphoreType.DMA,
            pltpu.SemaphoreType.DMA,
            pltpu.SemaphoreType.DMA,
        ],
        compiler_params=pltpu.CompilerParams(use_tc_tiling_on_sc=False,
                                             needs_layout_passes=False),
    )
    return f(ttb, tpd, ttok, i0, i1, i2, i3, i4)


def kernel(Tempo, Bar, Position, Token, Duration, tempo_table, bar_table,
           pos_table, token_table, dur_table, W_dec, b_dec):
    ttb, tpd, ttok = _project(tempo_table, bar_table, pos_table,
                              token_table, dur_table, W_dec, b_dec)
    shp = (N // IG, IG)
    out = _lookup(
        ttb, tpd, ttok,
        Tempo.reshape(shp), Bar.reshape(shp), Position.reshape(shp),
        Token.reshape(shp), Duration.reshape(shp),
    )
    return out.transpose(0, 2, 1)


# speedup vs baseline: 3.1911x; 1.0637x over previous
import jax
import jax.numpy as jnp
from jax import lax
from jax.experimental import pallas as pl
from jax.experimental.pallas import tpu as pltpu
from jax.experimental.pallas import tpu_sc as plsc

D = 64
B, L = 64, 2048
N = B * L
N_T, N_B, N_P, N_K, N_D = 64, 128, 128, 256, 128

NC, NS = 2, 16
NW = NC * NS
TPW = N // NW
IG = 128
CHUNK = 256
NG = CHUNK // IG
NCHUNK = TPW // CHUNK
CPB = L // CHUNK


def _project_body(tt, bt, pt, kt, dt, w, b, otb, opd, otok):
    bias = b[0, :]
    p_t = jnp.dot(tt[...], w[0:64, :], preferred_element_type=jnp.float32) + bias
    p_b = jnp.dot(bt[...], w[64:128, :], preferred_element_type=jnp.float32)
    p_p = jnp.dot(pt[...], w[128:192, :], preferred_element_type=jnp.float32)
    p_k = jnp.dot(kt[...], w[192:256, :], preferred_element_type=jnp.float32)
    p_d = jnp.dot(dt[...], w[256:320, :], preferred_element_type=jnp.float32)
    for i in range(N_T):
        otb[pl.ds(i * N_B, N_B), :] = p_b + p_t[i:i + 1, :]
    for i in range(N_P):
        opd[pl.ds(i * N_D, N_D), :] = p_d + p_p[i:i + 1, :]
    otok[...] = p_k


def _project(tt, bt, pt, kt, dt, w, b):
    return pl.pallas_call(
        _project_body,
        out_shape=[
            jax.ShapeDtypeStruct((N_T * N_B, D), jnp.float32),
            jax.ShapeDtypeStruct((N_P * N_D, D), jnp.float32),
            jax.ShapeDtypeStruct((N_K, D), jnp.float32),
        ],
    )(tt, bt, pt, kt, dt, w, b.reshape(1, D))


def _lookup_body(ttb, tpd, ttok, i0, i1, i2, i3, i4, out,
                 vi0, vi1, ci0, ci1, rg0, rg1, rb0, rb1, rc0, rc1,
                 ac0, sidx, sg, so):
    cid = lax.axis_index("c")
    sid = lax.axis_index("s")
    wid = sid * NC + cid
    row0 = wid * (TPW // IG)
    vi = (vi0, vi1)
    ci = (ci0, ci1)
    rg = (rg0, rg1)
    rb = (rb0, rb1)
    rc = (rc0, rc1)
    ac = (ac0, ac0)
    idx_arrs = (i0, i1, i2, i3, i4)

    def idx_cps(k, p):
        r = pl.ds(row0 + k * NG, NG)
        return [pltpu.make_async_copy(idx_arrs[f].at[r], vi[p].at[f], sidx)
                for f in range(5)]

    def gather_cps(k, p):
        cps = []
        for g in range(NG):
            dst = pl.ds(g * IG, IG)
            cps.append(pltpu.make_async_copy(
                ttb.at[ci[p].at[g, 0]], rg[p].at[dst], sg))
            cps.append(pltpu.make_async_copy(
                tpd.at[ci[p].at[g, 1]], rb[p].at[dst], sg))
            cps.append(pltpu.make_async_copy(
                ttok.at[vi[p].at[3, g]], rc[p].at[dst], sg))
        return cps

    def out_cp(k, p):
        bi = 2 * wid + (k // CPB)
        l0 = (k % CPB) * CHUNK
        return pltpu.make_async_copy(
            ac[p], out.at[bi, :, pl.ds(l0, CHUNK)], so)

    def combine(k, p):
        vip, cip = vi[p], ci[p]

        def cb(j, c):
            g = j // (IG // 16)
            col = (j % (IG // 16)) * 16
            sl = pl.ds(col, 16)
            cip[g, 0, sl] = vip[0, g, sl] * N_B + vip[1, g, sl]
            cip[g, 1, sl] = vip[2, g, sl] * N_D + vip[4, g, sl]
            return c
        lax.fori_loop(0, NG * (IG // 16), cb, 0)

    jbase = lax.broadcasted_iota(jnp.int32, (16,), 0)
    rots = [lax.rem(jbase + i, 16) for i in range(16)]
    NLT = CHUNK // 16

    def accum(p):
        rgp, rbp, rcp, acp = rg[p], rb[p], rc[p], ac[p]

        @plsc.parallel_loop(0, NLT * (D // 16), unroll=2)
        def ab(q):
            l0 = lax.rem(q, NLT) * 16
            d0 = lax.div(q, NLT) * 16
            didx = jbase + d0
            for i in range(16):
                ridx = rots[i] + l0
                v = (plsc.load_gather(rgp, [ridx, didx])
                     + plsc.load_gather(rbp, [ridx, didx])
                     + plsc.load_gather(rcp, [ridx, didx]))
                plsc.store_scatter(acp, [didx, ridx], v)

    def chunk_step(k, p):
        @pl.when(k + 1 < NCHUNK)
        def _():
            for cp in idx_cps(k + 1, p ^ 1):
                cp.wait()
            combine(k + 1, p ^ 1)
        for cp in gather_cps(k, p):
            cp.wait()

        @pl.when(k >= 1)
        def _():
            out_cp(k - 1, p ^ 1).wait()

        @pl.when(k + 1 < NCHUNK)
        def _():
            for cp in gather_cps(k + 1, p ^ 1):
                cp.start()

        @pl.when(k + 2 < NCHUNK)
        def _():
            for cp in idx_cps(k + 2, p):
                cp.start()
        accum(p)
        out_cp(k, p).start()

    for cp in idx_cps(0, 0):
        cp.start()
    for cp in idx_cps(0, 0):
        cp.wait()
    combine(0, 0)
    for cp in gather_cps(0, 0):
        cp.start()
    for cp in idx_cps(1, 1):
        cp.start()

    def steady(j, c):
        chunk_step(2 * j, 0)
        chunk_step(2 * j + 1, 1)
        return c
    lax.fori_loop(0, NCHUNK // 2, steady, 0)
    out_cp(NCHUNK - 1, 1).wait()


def _lookup(ttb, tpd, ttok, i0, i1, i2, i3, i4):
    mesh = plsc.VectorSubcoreMesh(core_axis_name="c", subcore_axis_name="s")
    f = pl.kernel(
        _lookup_body,
        out_type=jax.ShapeDtypeStruct((B, D, L), jnp.float32),
        mesh=mesh,
        scratch_types=[
            pltpu.VMEM((5, NG, IG), jnp.int32),
            pltpu.VMEM((5, NG, IG), jnp.int32),
            pltpu.VMEM((NG, 2, IG), jnp.int32),
            pltpu.VMEM((NG, 2, IG), jnp.int32),
            pltpu.VMEM((CHUNK, D), jnp.float32),
            pltpu.VMEM((CHUNK, D), jnp.float32),
            pltpu.VMEM((CHUNK, D), jnp.float32),
            pltpu.VMEM((CHUNK, D), jnp.float32),
            pltpu.VMEM((CHUNK, D), jnp.float32),
            pltpu.VMEM((CHUNK, D), jnp.float32),
            pltpu.VMEM((D, CHUNK), jnp.float32),
            pltpu.SemaphoreType.DMA,
            pltpu.SemaphoreType.DMA,
            pltpu.SemaphoreType.DMA,
        ],
        compiler_params=pltpu.CompilerParams(use_tc_tiling_on_sc=False,
                                             needs_layout_passes=False),
    )
    return f(ttb, tpd, ttok, i0, i1, i2, i3, i4)


def kernel(Tempo, Bar, Position, Token, Duration, tempo_table, bar_table,
           pos_table, token_table, dur_table, W_dec, b_dec):
    ttb, tpd, ttok = _project(tempo_table, bar_table, pos_table,
                              token_table, dur_table, W_dec, b_dec)
    shp = (N // IG, IG)
    out = _lookup(
        ttb, tpd, ttok,
        Tempo.reshape(shp), Bar.reshape(shp), Position.reshape(shp),
        Token.reshape(shp), Duration.reshape(shp),
    )
    return out.transpose(0, 2, 1)
